# Initial kernel scaffold; baseline (speedup 1.0000x reference)
import jax
import jax.numpy as jnp
from jax.experimental import pallas as pl


def kernel(x, edge_index, batch, W1, b1, gamma1, beta1, W2, b2, gamma2, beta2):
    raise NotImplementedError("write your pallas kernel here")


# speedup vs baseline: 18.3792x; 18.3792x over previous
import functools

import jax
import jax.numpy as jnp
from jax import lax
from jax.experimental import pallas as pl
from jax.experimental.pallas import tpu as pltpu
from jax.experimental.pallas import tpu_sc as plsc

N = 10000
E = 320000
D = 128
G = 64
EPS = 1e-5

NC = 2
NS = 16
NW = NC * NS
EPW = E // NW
K = 80
CH = EPW // K
NP = 10240
RPS = NP // NS
DEGW = 16


@functools.cache
def _sc_kernels():
    mesh = plsc.VectorSubcoreMesh(core_axis_name="c", subcore_axis_name="s",
                                  num_cores=NC, num_subcores=NS)

    @functools.partial(
        pl.kernel,
        out_type=jax.ShapeDtypeStruct((NC, NP, D), jnp.float32),
        mesh=mesh,
        scratch_types=[
            pltpu.VMEM((CH, K), jnp.int32),
            pltpu.VMEM((K, D), jnp.float32),
            pltpu.VMEM_SHARED((NP, D), jnp.float32),
        ],
    )
    def sc_deg(dst_hbm, ones_hbm, zagg_hbm, out_hbm, dst_v, ones_v, acc):
        c = lax.axis_index("c")
        s = lax.axis_index("s")
        wid = c * NS + s
        pltpu.sync_copy(dst_hbm.at[wid], dst_v)
        pltpu.sync_copy(ones_hbm, ones_v)
        pltpu.sync_copy(zagg_hbm, acc.at[pl.ds(s * RPS, RPS)])
        plsc.subcore_barrier()

        def step(j, carry):
            pltpu.sync_copy(ones_v, acc.at[dst_v.at[j]], add=True)
            return carry

        lax.fori_loop(0, CH, step, 0)
        plsc.subcore_barrier()
        pltpu.sync_copy(acc.at[pl.ds(s * RPS, RPS)],
                        out_hbm.at[c, pl.ds(s * RPS, RPS)])

    @functools.partial(
        pl.kernel,
        out_type=jax.ShapeDtypeStruct((NC, NP, D), jnp.float32),
        mesh=mesh,
        scratch_types=[
            pltpu.VMEM((CH, K), jnp.int32),
            pltpu.VMEM((CH, K), jnp.int32),
            pltpu.VMEM((K, D), jnp.float32),
            pltpu.SemaphoreType.DMA,
            pltpu.VMEM_SHARED((NP, D), jnp.float32),
        ],
    )
    def sc_agg(hs_hbm, src_hbm, dst_hbm, zagg_hbm, out_hbm,
               src_v, dst_v, rows_v, sem, acc):
        c = lax.axis_index("c")
        s = lax.axis_index("s")
        wid = c * NS + s
        pltpu.sync_copy(src_hbm.at[wid], src_v)
        pltpu.sync_copy(dst_hbm.at[wid], dst_v)
        pltpu.sync_copy(zagg_hbm, acc.at[pl.ds(s * RPS, RPS)])
        plsc.subcore_barrier()

        def step(j, carry):
            pltpu.async_copy(hs_hbm.at[src_v.at[j]], rows_v, sem).wait()
            pltpu.sync_copy(rows_v, acc.at[dst_v.at[j]], add=True)
            return carry

        lax.fori_loop(0, CH, step, 0)
        plsc.subcore_barrier()
        pltpu.sync_copy(acc.at[pl.ds(s * RPS, RPS)],
                        out_hbm.at[c, pl.ds(s * RPS, RPS)])

    return sc_deg, sc_agg


def _dis_from_deg(deg):
    d = deg[0, :N, :1] + deg[1, :N, :1] + 1.0
    return lax.rsqrt(jnp.maximum(d, 1.0))


def _tc_dense1(x_ref, w_ref, deg_ref, out_ref):
    dis = _dis_from_deg(deg_ref[...])
    h = jnp.dot(x_ref[...], w_ref[...], preferred_element_type=jnp.float32)
    out_ref[...] = h * dis


def _bn_relu(agg, hs, dis, b, gamma, beta):
    h = dis * (agg[0, :N] + agg[1, :N] + hs) + b
    mean = jnp.mean(h, axis=0, keepdims=True)
    var = jnp.mean((h - mean) ** 2, axis=0, keepdims=True)
    h = (h - mean) * lax.rsqrt(var + EPS) * gamma + beta
    return jnp.maximum(h, 0.0)


def _tc_dense2(agg_ref, hs_ref, deg_ref, b_ref, g_ref, be_ref, w_ref, out_ref):
    dis = _dis_from_deg(deg_ref[...])
    h = _bn_relu(agg_ref[...], hs_ref[...], dis, b_ref[...], g_ref[...],
                 be_ref[...])
    h2 = jnp.dot(h, w_ref[...], preferred_element_type=jnp.float32)
    out_ref[...] = h2 * dis


def _tc_dense3(agg_ref, hs_ref, deg_ref, b_ref, g_ref, be_ref, batch_ref,
               out_ref):
    dis = _dis_from_deg(deg_ref[...])
    h = _bn_relu(agg_ref[...], hs_ref[...], dis, b_ref[...], g_ref[...],
                 be_ref[...])
    seg = lax.broadcasted_iota(jnp.int32, (N, G), 1)
    onehot = (batch_ref[...] == seg).astype(jnp.float32)
    sums = lax.dot_general(onehot, h, (((0,), (0,)), ((), ())),
                           preferred_element_type=jnp.float32)
    counts = jnp.sum(onehot, axis=0)[:, None]
    out_ref[...] = sums / jnp.maximum(counts, 1.0)


def kernel(x, edge_index, batch, W1, b1, gamma1, beta1, W2, b2, gamma2, beta2):
    src = edge_index[0].reshape(NW, CH, K)
    dst = edge_index[1].reshape(NW, CH, K)
    batch2 = batch.reshape(N, 1)
    b1r, g1r, be1r = b1.reshape(1, D), gamma1.reshape(1, D), beta1.reshape(1, D)
    b2r, g2r, be2r = b2.reshape(1, D), gamma2.reshape(1, D), beta2.reshape(1, D)

    ones_deg = jnp.ones((K, D), jnp.float32)
    zagg = jnp.zeros((RPS, D), jnp.float32)

    sc_deg, sc_agg = _sc_kernels()
    deg = sc_deg(dst, ones_deg, zagg)

    hs1 = pl.pallas_call(
        _tc_dense1,
        out_shape=jax.ShapeDtypeStruct((N, D), jnp.float32),
    )(x, W1, deg)

    agg1 = sc_agg(hs1, src, dst, zagg)

    hs2 = pl.pallas_call(
        _tc_dense2,
        out_shape=jax.ShapeDtypeStruct((N, D), jnp.float32),
    )(agg1, hs1, deg, b1r, g1r, be1r, W2)

    agg2 = sc_agg(hs2, src, dst, zagg)

    out = pl.pallas_call(
        _tc_dense3,
        out_shape=jax.ShapeDtypeStruct((G, D), jnp.float32),
    )(agg2, hs2, deg, b2r, g2r, be2r, batch2)

    return out
